# Initial kernel scaffold; baseline (speedup 1.0000x reference)
import jax
import jax.numpy as jnp
from jax.experimental import pallas as pl


def kernel(x_rows, x_cols, x_vals, edge, adj_vals, gene_p, W, a):
    raise NotImplementedError("write your pallas kernel here")


# speedup vs baseline: 1.0251x; 1.0251x over previous
import jax
import jax.numpy as jnp
from jax import lax
from jax.experimental import pallas as pl
from jax.experimental.pallas import tpu as pltpu

N = 10000
D_IN = 128
D_OUT = 128
E = 320000
NNZ = 128000

_BM = 1000


def _matmul_body(x_ref, w_ref, o_ref):
    o_ref[...] = jnp.dot(x_ref[...], w_ref[...],
                         preferred_element_type=jnp.float32)


def _matmul(x_dense, W):
    return pl.pallas_call(
        _matmul_body,
        grid=(N // _BM,),
        in_specs=[
            pl.BlockSpec((_BM, D_IN), lambda i: (i, 0)),
            pl.BlockSpec((D_IN, D_OUT), lambda i: (0, 0)),
        ],
        out_specs=pl.BlockSpec((_BM, D_OUT), lambda i: (i, 0)),
        out_shape=jax.ShapeDtypeStruct((N, D_OUT), jnp.float32),
    )(x_dense, W)


def kernel(x_rows, x_cols, x_vals, edge, adj_vals, gene_p, W, a):
    x_dense = jnp.zeros((N, D_IN), dtype=jnp.float32).at[x_rows, x_cols].add(x_vals)
    h = _matmul(x_dense, W)
    gp = gene_p[:, None]
    src = edge[0]
    dst = edge[1]
    edge_v = jnp.abs(h[src] - h[dst])
    edge_v_p = jnp.abs(gp[src] - gp[dst])
    edge_v = edge_v * edge_v_p
    edge_v = jnp.squeeze(jax.nn.relu(edge_v @ a))
    edge_v = jnp.exp(edge_v) * jnp.power(adj_vals, 25)
    return (h, edge_v)


# speedup vs baseline: 7.6042x; 7.4180x over previous
import functools

import jax
import jax.numpy as jnp
from jax import lax
from jax.experimental import pallas as pl
from jax.experimental.pallas import tpu as pltpu
from jax.experimental.pallas import tpu_sc as plsc

N = 10000
D_IN = 128
D_OUT = 128
E = 320000
NNZ = 128000

_BM = 1000


def _matmul_body(x_ref, w_ref, o_ref):
    o_ref[...] = jnp.dot(x_ref[...], w_ref[...],
                         preferred_element_type=jnp.float32)


def _matmul(x_dense, W):
    return pl.pallas_call(
        _matmul_body,
        grid=(N // _BM,),
        in_specs=[
            pl.BlockSpec((_BM, D_IN), lambda i: (i, 0)),
            pl.BlockSpec((D_IN, D_OUT), lambda i: (0, 0)),
        ],
        out_specs=pl.BlockSpec((_BM, D_OUT), lambda i: (i, 0)),
        out_shape=jax.ShapeDtypeStruct((N, D_OUT), jnp.float32),
    )(x_dense, W)


_NC = 2
_NS = 16
_NW = _NC * _NS
_EPW = E // _NW
_EC = 80
_NCHUNK = _EPW // _EC
_L = 16


def _edge_body(h_hbm, src_hbm, dst_hbm, adj_hbm, gp_hbm, a_hbm, out_hbm,
               src_v, dst_v, adj_v, gp_v, a_v, out_v, hs_v, hd_v, t2_v, sem):
    wid = lax.axis_index("s") * _NC + lax.axis_index("c")
    base = wid * _EPW
    pltpu.sync_copy(src_hbm.at[pl.ds(base, _EPW)], src_v)
    pltpu.sync_copy(dst_hbm.at[pl.ds(base, _EPW)], dst_v)
    pltpu.sync_copy(adj_hbm.at[pl.ds(base, _EPW)], adj_v)
    pltpu.sync_copy(gp_hbm, gp_v)
    pltpu.sync_copy(a_hbm, a_v)

    lane = lax.iota(jnp.int32, _L)

    n_grp = _EC // _L
    lane16 = lane * _L
    a_regs = [a_v[pl.ds(jb * _L, _L)] for jb in range(D_OUT // _L)]

    def chunk_body(c, carry):
        off = c * _EC
        cp1 = pltpu.async_copy(h_hbm.at[src_v.at[pl.ds(off, _EC)]], hs_v, sem)
        cp2 = pltpu.async_copy(h_hbm.at[dst_v.at[pl.ds(off, _EC)]], hd_v, sem)
        cp1.wait()
        cp2.wait()
        for g in range(n_grp):
            for e in range(_L):
                row = g * _L + e
                acc = jnp.zeros((_L,), jnp.float32)
                for j in range(D_OUT // _L):
                    hs = hs_v[row, pl.ds(j * _L, _L)]
                    hd = hd_v[row, pl.ds(j * _L, _L)]
                    acc = acc + jnp.abs(hs - hd) * a_regs[j]
                t2_v[pl.ds(e * _L, _L)] = acc
            tv = jnp.zeros((_L,), jnp.float32)
            for col in range(_L):
                tv = tv + plsc.load_gather(t2_v, [lane16 + col])
            sl = pl.ds(off + g * _L, _L)
            gs = plsc.load_gather(gp_v, [src_v[sl]])
            gd = plsc.load_gather(gp_v, [dst_v[sl]])
            u = jnp.maximum(tv * jnp.abs(gs - gd), 0.0)
            ev = jnp.exp(u)
            ad = adj_v[sl]
            a2 = ad * ad
            a4 = a2 * a2
            a8 = a4 * a4
            a16 = a8 * a8
            out_v[sl] = ev * (a16 * a8 * ad)
        return carry

    lax.fori_loop(0, _NCHUNK, chunk_body, 0)
    pltpu.sync_copy(out_v, out_hbm.at[pl.ds(base, _EPW)])


def _edge_stage(h, src, dst, adj_vals, gene_p, a_flat):
    mesh = plsc.VectorSubcoreMesh(core_axis_name="c", subcore_axis_name="s")
    f = functools.partial(
        pl.kernel,
        out_type=jax.ShapeDtypeStruct((E,), jnp.float32),
        mesh=mesh,
        compiler_params=pltpu.CompilerParams(needs_layout_passes=False),
        scratch_types=[
            pltpu.VMEM((_EPW,), jnp.int32),
            pltpu.VMEM((_EPW,), jnp.int32),
            pltpu.VMEM((_EPW,), jnp.float32),
            pltpu.VMEM((N,), jnp.float32),
            pltpu.VMEM((D_OUT,), jnp.float32),
            pltpu.VMEM((_EPW,), jnp.float32),
            pltpu.VMEM((_EC, D_OUT), jnp.float32),
            pltpu.VMEM((_EC, D_OUT), jnp.float32),
            pltpu.VMEM((_L * _L,), jnp.float32),
            pltpu.SemaphoreType.DMA,
        ],
    )(_edge_body)
    return f(h, src, dst, adj_vals, gene_p, a_flat)


def kernel(x_rows, x_cols, x_vals, edge, adj_vals, gene_p, W, a):
    x_dense = jnp.zeros((N, D_IN), dtype=jnp.float32).at[x_rows, x_cols].add(x_vals)
    h = _matmul(x_dense, W)
    src = edge[0].astype(jnp.int32)
    dst = edge[1].astype(jnp.int32)
    edge_v = _edge_stage(h, src, dst, adj_vals.astype(jnp.float32),
                         gene_p.astype(jnp.float32),
                         a.reshape(D_OUT).astype(jnp.float32))
    return (h, edge_v)
